# Initial kernel scaffold; baseline (speedup 1.0000x reference)
import jax
import jax.numpy as jnp
from jax import lax
from jax.experimental import pallas as pl
from jax.experimental.pallas import tpu as pltpu
from jax.experimental.pallas import tpu_sc as plsc

N = 32768
C = 2048
NW = 32
RPW = N // NW
L = 16
G = C // L


def _body(x_hbm, m_hbm, o_hbm, mvm, xvm, ovm, sem_in, sem_out):
    wid = lax.axis_index("s") * 2 + lax.axis_index("c")
    base = wid * RPW
    pltpu.sync_copy(m_hbm.at[pl.ds(base, RPW)], mvm)
    lanes = lax.iota(jnp.int32, L)

    def row(r, carry):
        i = base + r
        pltpu.sync_copy(x_hbm.at[i], xvm)
        mm = mvm[r]

        def gran(g, _):
            idx = lanes + (mm + g * L)
            idx = jnp.where(idx >= C, idx - C, idx)
            ovm[pl.ds(g * L, L)] = plsc.load_gather(xvm, [idx])
            return 0

        lax.fori_loop(0, G, gran, 0, unroll=4)
        pltpu.sync_copy(ovm, o_hbm.at[i])
        return carry

    lax.fori_loop(0, RPW, row, 0)


def kernel(x, map):
    m = map.reshape(-1).astype(jnp.int32)
    mesh = plsc.VectorSubcoreMesh(core_axis_name="c", subcore_axis_name="s")
    f = pl.kernel(
        _body,
        out_type=jax.ShapeDtypeStruct((N, C), jnp.float32),
        mesh=mesh,
        scratch_types=[
            pltpu.VMEM((RPW,), jnp.int32),
            pltpu.VMEM((C,), jnp.float32),
            pltpu.VMEM((C,), jnp.float32),
            pltpu.SemaphoreType.DMA,
            pltpu.SemaphoreType.DMA,
        ],
    )
    return f(x, m)


# speedup vs baseline: 3.3890x; 3.3890x over previous
import jax
import jax.numpy as jnp
from jax import lax
from jax.experimental import pallas as pl
from jax.experimental.pallas import tpu as pltpu
from jax.experimental.pallas import tpu_sc as plsc

N = 32768
C = 2048
NW = 32
RPW = N // NW
L = 16
G = C // L


def _body(x_hbm, m_hbm, o_hbm, mvm, xvm, ovm, sem_in, sem_out):
    wid = lax.axis_index("s") * 2 + lax.axis_index("c")
    base = wid * RPW
    pltpu.sync_copy(m_hbm.at[pl.ds(base, RPW)], mvm.at[pl.ds(0, RPW)])
    lanes = lax.iota(jnp.int32, L)

    def row(r, carry):
        i = base + r
        pltpu.sync_copy(x_hbm.at[i], xvm)
        mm = mvm[pl.ds(r, L)][0]

        def gran(g, _):
            idx = lanes + (mm + g * L)
            idx = jnp.where(idx >= C, idx - C, idx)
            ovm[pl.ds(g * L, L)] = plsc.load_gather(xvm, [idx])
            return 0

        lax.fori_loop(0, G, gran, 0, unroll=4)
        pltpu.sync_copy(ovm, o_hbm.at[i])
        return carry

    lax.fori_loop(0, RPW, row, 0)


def kernel(x, map):
    m = map.reshape(-1).astype(jnp.int32)
    mesh = plsc.VectorSubcoreMesh(core_axis_name="c", subcore_axis_name="s")
    f = pl.kernel(
        _body,
        out_type=jax.ShapeDtypeStruct((N, C), jnp.float32),
        mesh=mesh,
        scratch_types=[
            pltpu.VMEM((RPW + L,), jnp.int32),
            pltpu.VMEM((C,), jnp.float32),
            pltpu.VMEM((C,), jnp.float32),
            pltpu.SemaphoreType.DMA,
            pltpu.SemaphoreType.DMA,
        ],
        compiler_params=pltpu.CompilerParams(needs_layout_passes=False),
    )
    return f(x, m)


# speedup vs baseline: 4.9213x; 1.4522x over previous
import jax
import jax.numpy as jnp
from jax import lax
from jax.experimental import pallas as pl
from jax.experimental.pallas import tpu as pltpu
from jax.experimental.pallas import tpu_sc as plsc

N = 32768
C = 2048
NW = 32
RPW = N // NW
L = 16
G = C // L
B = 8
NBAT = RPW // B


def _body(x_hbm, m_hbm, o_hbm, mvm, xb0, xb1, ob0, ob1,
          sin0, sin1, sout0, sout1):
    wid = lax.axis_index("s") * 2 + lax.axis_index("c")
    base = wid * RPW
    pltpu.sync_copy(m_hbm.at[pl.ds(base, RPW)], mvm.at[pl.ds(0, RPW)])
    lanes = lax.iota(jnp.int32, L)
    xbufs = (xb0, xb1)
    obufs = (ob0, ob1)
    sins = (sin0, sin1)
    souts = (sout0, sout1)

    def start_in(b, k):
        rs = (base + b * B) * C
        pltpu.async_copy(x_hbm.at[pl.ds(rs, B * C)], xbufs[k], sins[k])

    def wait_in(k):
        pltpu.make_async_copy(
            x_hbm.at[pl.ds(0, B * C)], xbufs[k], sins[k]).wait()

    def start_out(b, k):
        rs = (base + b * B) * C
        pltpu.async_copy(obufs[k], o_hbm.at[pl.ds(rs, B * C)], souts[k])

    def wait_out(k):
        pltpu.make_async_copy(
            obufs[k], o_hbm.at[pl.ds(0, B * C)], souts[k]).wait()

    def do_batch(b, k):
        xbk = xbufs[k]
        obk = obufs[k]
        mvec = mvm[pl.ds(b * B, L)]
        for rb in range(B):
            mm = mvec[rb]
            xrow = xbk.at[pl.ds(rb * C, C)]
            orow = obk.at[pl.ds(rb * C, C)]
            idx_e = (lanes + mm) & (C - 1)
            idx_o = (idx_e + L) & (C - 1)

            def gran(h, carry):
                ie, io = carry
                orow[pl.ds((2 * h) * L, L)] = plsc.load_gather(xrow, [ie])
                orow[pl.ds((2 * h + 1) * L, L)] = plsc.load_gather(xrow, [io])
                return ((ie + 2 * L) & (C - 1), (io + 2 * L) & (C - 1))

            lax.fori_loop(0, G // 2, gran, (idx_e, idx_o), unroll=4)

    start_in(0, 0)

    def outer(bb, carry):
        for k in range(2):
            b = 2 * bb + k

            @pl.when(b + 1 < NBAT)
            def _():
                start_in(b + 1, 1 - k)

            wait_in(k)

            @pl.when(b >= 2)
            def _():
                wait_out(k)

            do_batch(b, k)
            start_out(b, k)
        return carry

    lax.fori_loop(0, NBAT // 2, outer, 0)
    wait_out(0)
    wait_out(1)


def kernel(x, map):
    m = map.reshape(-1).astype(jnp.int32)
    xf = x.reshape(-1)
    mesh = plsc.VectorSubcoreMesh(core_axis_name="c", subcore_axis_name="s")
    f = pl.kernel(
        _body,
        out_type=jax.ShapeDtypeStruct((N * C,), jnp.float32),
        mesh=mesh,
        scratch_types=[
            pltpu.VMEM((RPW + L,), jnp.int32),
            pltpu.VMEM((B * C,), jnp.float32),
            pltpu.VMEM((B * C,), jnp.float32),
            pltpu.VMEM((B * C,), jnp.float32),
            pltpu.VMEM((B * C,), jnp.float32),
            pltpu.SemaphoreType.DMA,
            pltpu.SemaphoreType.DMA,
            pltpu.SemaphoreType.DMA,
            pltpu.SemaphoreType.DMA,
        ],
        compiler_params=pltpu.CompilerParams(needs_layout_passes=False),
    )
    return f(xf, m).reshape(N, C)


# speedup vs baseline: 8.7298x; 1.7739x over previous
import jax
import jax.numpy as jnp
from jax import lax
from jax.experimental import pallas as pl
from jax.experimental.pallas import tpu as pltpu
from jax.experimental.pallas import tpu_sc as plsc

N = 32768
C = 2048
NW = 32
RPW = N // NW
L = 16
G = C // L
B = 8
NBAT = RPW // B


def _body(x_hbm, m_hbm, o_hbm, mvm, xb0, xb1, ob0, ob1,
          sin0, sin1, sout0, sout1):
    wid = lax.axis_index("s") * 2 + lax.axis_index("c")
    base = wid * RPW
    pltpu.sync_copy(m_hbm.at[pl.ds(base, RPW)], mvm.at[pl.ds(0, RPW)])
    lanes = lax.iota(jnp.int32, L)
    xbufs = (xb0, xb1)
    obufs = (ob0, ob1)
    sins = (sin0, sin1)
    souts = (sout0, sout1)

    def start_in(b, k):
        rs = base + b * B
        pltpu.async_copy(x_hbm.at[pl.ds(rs, B)], xbufs[k], sins[k])

    def wait_in(k):
        pltpu.make_async_copy(x_hbm.at[pl.ds(0, B)], xbufs[k], sins[k]).wait()

    def start_out(b, k):
        rs = base + b * B
        pltpu.async_copy(obufs[k], o_hbm.at[pl.ds(rs, B)], souts[k])

    def wait_out(k):
        pltpu.make_async_copy(obufs[k], o_hbm.at[pl.ds(0, B)], souts[k]).wait()

    def do_batch(b, k):
        xbk = xbufs[k]
        obk = obufs[k]
        mvec = mvm[pl.ds(b * B, L)]
        for rb in range(B):
            mm = mvec[rb]
            rvec = lanes * 0 + rb
            idx_e = (lanes + mm) & (C - 1)
            idx_o = (idx_e + L) & (C - 1)

            def gran(h, carry):
                ie, io = carry
                obk[rb, pl.ds((2 * h) * L, L)] = plsc.load_gather(
                    xbk, [rvec, ie])
                obk[rb, pl.ds((2 * h + 1) * L, L)] = plsc.load_gather(
                    xbk, [rvec, io])
                return ((ie + 2 * L) & (C - 1), (io + 2 * L) & (C - 1))

            lax.fori_loop(0, G // 2, gran, (idx_e, idx_o), unroll=4)

    start_in(0, 0)

    def outer(bb, carry):
        for k in range(2):
            b = 2 * bb + k

            @pl.when(b + 1 < NBAT)
            def _():
                start_in(b + 1, 1 - k)

            wait_in(k)

            @pl.when(b >= 2)
            def _():
                wait_out(k)

            do_batch(b, k)
            start_out(b, k)
        return carry

    lax.fori_loop(0, NBAT // 2, outer, 0)
    wait_out(0)
    wait_out(1)


def kernel(x, map):
    m = map.reshape(-1).astype(jnp.int32)
    mesh = plsc.VectorSubcoreMesh(core_axis_name="c", subcore_axis_name="s")
    f = pl.kernel(
        _body,
        out_type=jax.ShapeDtypeStruct((N, C), jnp.float32),
        mesh=mesh,
        scratch_types=[
            pltpu.VMEM((RPW + L,), jnp.int32),
            pltpu.VMEM((B, C), jnp.float32),
            pltpu.VMEM((B, C), jnp.float32),
            pltpu.VMEM((B, C), jnp.float32),
            pltpu.VMEM((B, C), jnp.float32),
            pltpu.SemaphoreType.DMA,
            pltpu.SemaphoreType.DMA,
            pltpu.SemaphoreType.DMA,
            pltpu.SemaphoreType.DMA,
        ],
        compiler_params=pltpu.CompilerParams(needs_layout_passes=False),
    )
    return f(x, m)


# speedup vs baseline: 17.2370x; 1.9745x over previous
import jax
import jax.numpy as jnp
from jax import lax
from jax.experimental import pallas as pl
from jax.experimental.pallas import tpu as pltpu
from jax.experimental.pallas import tpu_sc as plsc

N = 32768
C = 2048
NW = 32
RPW = N // NW
L = 16
G = C // L
B = 8
NBAT = RPW // B
NCH = 8
NT = C // (NCH * L)


def _body(x_hbm, m_hbm, o_hbm, mvm, xb0, xb1, ob0, ob1,
          sin0, sin1, sout0, sout1):
    wid = lax.axis_index("s") * 2 + lax.axis_index("c")
    base = wid * RPW
    pltpu.sync_copy(m_hbm.at[pl.ds(base, RPW)], mvm.at[pl.ds(0, RPW)])
    lanes = lax.iota(jnp.int32, L)
    xbufs = (xb0, xb1)
    obufs = (ob0, ob1)
    sins = (sin0, sin1)
    souts = (sout0, sout1)

    def start_in(b, k):
        rs = base + b * B
        for rb in range(B):
            pltpu.async_copy(
                x_hbm.at[rs + rb], xbufs[k].at[pl.ds(rb * C, C)], sins[k])

    def wait_in(k):
        for rb in range(B):
            pltpu.make_async_copy(
                x_hbm.at[0], xbufs[k].at[pl.ds(rb * C, C)], sins[k]).wait()

    def start_out(b, k):
        rs = base + b * B
        pltpu.async_copy(obufs[k], o_hbm.at[pl.ds(rs, B)], souts[k])

    def wait_out(k):
        pltpu.make_async_copy(obufs[k], o_hbm.at[pl.ds(0, B)], souts[k]).wait()

    def do_batch(b, k):
        xbk = xbufs[k]
        obk = obufs[k]
        mvec = mvm[pl.ds(b * B, L)]
        for rb in range(B):
            mm = mvec[rb]
            xrow = xbk.at[pl.ds(rb * C, C)]

            def gath(vs):
                return tuple(plsc.load_gather(xrow, [v]) for v in vs)

            def adv(vs):
                return tuple((v + NCH * L) & (C - 1) for v in vs)

            vs = tuple(
                (lanes + (mm + o * L)) & (C - 1) for o in range(NCH))
            vals = gath(vs)
            vs = adv(vs)

            def tstep(t, carry):
                vs, vals = carry
                col = t * (NCH * L)
                for o in range(NCH):
                    obk[rb, pl.ds(col + o * L, L)] = vals[o]
                return adv(vs), gath(vs)

            lax.fori_loop(0, NT, tstep, (vs, vals), unroll=4)

    start_in(0, 0)

    def outer(bb, carry):
        for k in range(2):
            b = 2 * bb + k

            @pl.when(b + 1 < NBAT)
            def _():
                start_in(b + 1, 1 - k)

            wait_in(k)

            @pl.when(b >= 2)
            def _():
                wait_out(k)

            do_batch(b, k)
            start_out(b, k)
        return carry

    lax.fori_loop(0, NBAT // 2, outer, 0)
    wait_out(0)
    wait_out(1)


def kernel(x, map):
    m = map.reshape(-1).astype(jnp.int32)
    mesh = plsc.VectorSubcoreMesh(core_axis_name="c", subcore_axis_name="s")
    f = pl.kernel(
        _body,
        out_type=jax.ShapeDtypeStruct((N, C), jnp.float32),
        mesh=mesh,
        scratch_types=[
            pltpu.VMEM((RPW + L,), jnp.int32),
            pltpu.VMEM((B * C,), jnp.float32),
            pltpu.VMEM((B * C,), jnp.float32),
            pltpu.VMEM((B, C), jnp.float32),
            pltpu.VMEM((B, C), jnp.float32),
            pltpu.SemaphoreType.DMA,
            pltpu.SemaphoreType.DMA,
            pltpu.SemaphoreType.DMA,
            pltpu.SemaphoreType.DMA,
        ],
        compiler_params=pltpu.CompilerParams(needs_layout_passes=False),
    )
    return f(x, m)


# speedup vs baseline: 19.3056x; 1.1200x over previous
import jax
import jax.numpy as jnp
from jax import lax
from jax.experimental import pallas as pl
from jax.experimental.pallas import tpu as pltpu
from jax.experimental.pallas import tpu_sc as plsc

N = 32768
C = 2048
NW = 32
RPW = N // NW
L = 16
G = C // L
B = 8
NBAT = RPW // B
NCH = 8
NT = C // (NCH * L)


def _body(x_hbm, m_hbm, o_hbm, mvm, xb0, xb1, ob0, ob1,
          sin0, sin1, sout0, sout1):
    wid = lax.axis_index("s") * 2 + lax.axis_index("c")
    base = wid * RPW
    pltpu.sync_copy(m_hbm.at[pl.ds(base, RPW)], mvm.at[pl.ds(0, RPW)])
    lanes = lax.iota(jnp.int32, L)
    xbufs = (xb0, xb1)
    obufs = (ob0, ob1)
    sins = (sin0, sin1)
    souts = (sout0, sout1)

    def start_in(b, k):
        rs = base + b * B
        pltpu.async_copy(x_hbm.at[pl.ds(rs, B)], xbufs[k], sins[k])

    def wait_in(k):
        pltpu.make_async_copy(x_hbm.at[pl.ds(0, B)], xbufs[k], sins[k]).wait()

    def start_out(b, k):
        rs = base + b * B
        pltpu.async_copy(obufs[k], o_hbm.at[pl.ds(rs, B)], souts[k])

    def wait_out(k):
        pltpu.make_async_copy(obufs[k], o_hbm.at[pl.ds(0, B)], souts[k]).wait()

    def do_batch(b, k):
        xbk = xbufs[k]
        obk = obufs[k]
        mvec = mvm[pl.ds(b * B, L)]
        for rb in range(B):
            mm = mvec[rb]
            rvec = lanes * 0 + rb

            def gath(vs):
                return tuple(plsc.load_gather(xbk, [rvec, v]) for v in vs)

            def adv(vs):
                return tuple((v + NCH * L) & (C - 1) for v in vs)

            vs = tuple(
                (lanes + (mm + o * L)) & (C - 1) for o in range(NCH))
            vals = gath(vs)
            vs = adv(vs)

            def tstep(t, carry):
                vs, vals = carry
                col = t * (NCH * L)
                for o in range(NCH):
                    obk[rb, pl.ds(col + o * L, L)] = vals[o]
                return adv(vs), gath(vs)

            lax.fori_loop(0, NT, tstep, (vs, vals), unroll=4)

    start_in(0, 0)

    def outer(bb, carry):
        for k in range(2):
            b = 2 * bb + k

            @pl.when(b + 1 < NBAT)
            def _():
                start_in(b + 1, 1 - k)

            wait_in(k)

            @pl.when(b >= 2)
            def _():
                wait_out(k)

            do_batch(b, k)
            start_out(b, k)
        return carry

    lax.fori_loop(0, NBAT // 2, outer, 0)
    wait_out(0)
    wait_out(1)


def kernel(x, map):
    m = map.reshape(-1).astype(jnp.int32)
    mesh = plsc.VectorSubcoreMesh(core_axis_name="c", subcore_axis_name="s")
    f = pl.kernel(
        _body,
        out_type=jax.ShapeDtypeStruct((N, C), jnp.float32),
        mesh=mesh,
        scratch_types=[
            pltpu.VMEM((RPW + L,), jnp.int32),
            pltpu.VMEM((B, C), jnp.float32),
            pltpu.VMEM((B, C), jnp.float32),
            pltpu.VMEM((B, C), jnp.float32),
            pltpu.VMEM((B, C), jnp.float32),
            pltpu.SemaphoreType.DMA,
            pltpu.SemaphoreType.DMA,
            pltpu.SemaphoreType.DMA,
            pltpu.SemaphoreType.DMA,
        ],
        compiler_params=pltpu.CompilerParams(needs_layout_passes=False),
    )
    return f(x, m)


# speedup vs baseline: 21.1519x; 1.0956x over previous
import jax
import jax.numpy as jnp
from jax import lax
from jax.experimental import pallas as pl
from jax.experimental.pallas import tpu as pltpu
from jax.experimental.pallas import tpu_sc as plsc

N = 32768
C = 2048
NW = 32
RPW = N // NW
L = 16
G = C // L
B = 4
NBUF = 4
NBAT = RPW // B
NCH = 8
NT = C // (NCH * L)


def _body(x_hbm, m_hbm, o_hbm, mvm,
          xb0, xb1, xb2, xb3, ob0, ob1, ob2, ob3,
          sin0, sin1, sin2, sin3, sout0, sout1, sout2, sout3):
    wid = lax.axis_index("s") * 2 + lax.axis_index("c")
    base = wid * RPW
    pltpu.sync_copy(m_hbm.at[pl.ds(base, RPW)], mvm.at[pl.ds(0, RPW)])
    lanes = lax.iota(jnp.int32, L)
    xbufs = (xb0, xb1, xb2, xb3)
    obufs = (ob0, ob1, ob2, ob3)
    sins = (sin0, sin1, sin2, sin3)
    souts = (sout0, sout1, sout2, sout3)

    def start_in(b, k):
        rs = base + b * B
        pltpu.async_copy(x_hbm.at[pl.ds(rs, B)], xbufs[k], sins[k])

    def wait_in(k):
        pltpu.make_async_copy(x_hbm.at[pl.ds(0, B)], xbufs[k], sins[k]).wait()

    def start_out(b, k):
        rs = base + b * B
        pltpu.async_copy(obufs[k], o_hbm.at[pl.ds(rs, B)], souts[k])

    def wait_out(k):
        pltpu.make_async_copy(obufs[k], o_hbm.at[pl.ds(0, B)], souts[k]).wait()

    def do_batch(b, k):
        xbk = xbufs[k]
        obk = obufs[k]
        mvec = mvm[pl.ds(b * B, L)]
        for rb in range(B):
            mm = mvec[rb]
            rvec = lanes * 0 + rb

            def gath(vs):
                return tuple(plsc.load_gather(xbk, [rvec, v]) for v in vs)

            def adv(vs):
                return tuple((v + NCH * L) & (C - 1) for v in vs)

            vs = tuple(
                (lanes + (mm + o * L)) & (C - 1) for o in range(NCH))
            vals = gath(vs)
            vs = adv(vs)

            def tstep(t, carry):
                vs, vals = carry
                col = t * (NCH * L)
                for o in range(NCH):
                    obk[rb, pl.ds(col + o * L, L)] = vals[o]
                return adv(vs), gath(vs)

            lax.fori_loop(0, NT, tstep, (vs, vals), unroll=4)

    for j in range(NBUF - 1):
        start_in(j, j)

    def outer(bb, carry):
        for k in range(NBUF):
            b = NBUF * bb + k
            nb = b + NBUF - 1
            nk = (k + NBUF - 1) % NBUF

            @pl.when(nb < NBAT)
            def _():
                start_in(nb, nk)

            wait_in(k)

            @pl.when(b >= NBUF)
            def _():
                wait_out(k)

            do_batch(b, k)
            start_out(b, k)
        return carry

    lax.fori_loop(0, NBAT // NBUF, outer, 0)
    for k in range(NBUF):
        wait_out(k)


def kernel(x, map):
    m = map.reshape(-1).astype(jnp.int32)
    mesh = plsc.VectorSubcoreMesh(core_axis_name="c", subcore_axis_name="s")
    f = pl.kernel(
        _body,
        out_type=jax.ShapeDtypeStruct((N, C), jnp.float32),
        mesh=mesh,
        scratch_types=(
            [pltpu.VMEM((RPW + L,), jnp.int32)]
            + [pltpu.VMEM((B, C), jnp.float32) for _ in range(2 * NBUF)]
            + [pltpu.SemaphoreType.DMA for _ in range(2 * NBUF)]
        ),
        compiler_params=pltpu.CompilerParams(needs_layout_passes=False),
    )
    return f(x, m)
